# baseline (speedup 1.0000x reference)
import jax
import jax.numpy as jnp
from jax.experimental import pallas as pl
from jax.experimental.pallas import tpu as pltpu

D = 768
K = 5
TQ = 256
TM = 4096
LW = 128


def _cmp2(a, b):
    return jnp.minimum(a, b), jnp.maximum(a, b)


def _oem(A, B):
    A = list(A)
    B = list(B)
    if not A:
        return B
    if not B:
        return A
    if len(A) == 1 and len(B) == 1:
        lo, hi = _cmp2(A[0], B[0])
        return [lo, hi]
    O = _oem(A[0::2], B[0::2])
    E = _oem(A[1::2], B[1::2])
    res = [O[0]]
    for i in range(len(E)):
        if i + 1 < len(O):
            lo, hi = _cmp2(E[i], O[i + 1])
            res += [lo, hi]
        else:
            res.append(E[i])
    res += O[len(E) + 1:]
    return res


def _msort(L):
    L = list(L)
    if len(L) <= 1:
        return L
    h = len(L) // 2
    return _oem(_msort(L[:h]), _msort(L[h:]))


def _bottomk(L, k):
    if len(L) <= k:
        return _msort(L)
    h = len(L) // 2
    return _oem(_bottomk(L[:h], k), _bottomk(L[h:], k))[:k]


def _body(x_ref, w1_ref, b1_ref, w2_ref, b2_ref, g_ref, beta_ref, pm_ref,
          o_ref, qn2_s, qsq_s, memb_s, msq_s, cand_s):
    mi = pl.program_id(0)
    qi = pl.program_id(1)
    nm1 = pl.num_programs(0)
    inf = jnp.float32(jnp.inf)
    rows = pl.ds(qi * TQ, TQ)

    @pl.when(mi == 0)
    def _mlp():
        x = x_ref[...].astype(jnp.bfloat16)
        h = jnp.maximum(
            jax.lax.dot_general(x, w1_ref[...], (((1,), (0,)), ((), ())),
                                preferred_element_type=jnp.float32)
            + b1_ref[...], 0.0)
        f = jax.lax.dot_general(h.astype(jnp.bfloat16), w2_ref[...],
                                (((1,), (0,)), ((), ())),
                                preferred_element_type=jnp.float32) + b2_ref[...]
        mu = jnp.mean(f, axis=-1, keepdims=True)
        var = jnp.mean((f - mu) * (f - mu), axis=-1, keepdims=True)
        q = (f - mu) * jax.lax.rsqrt(var + 1e-5) * g_ref[...] + beta_ref[...]
        qn2_s[rows, :] = (-2.0 * q).astype(jnp.bfloat16)
        qsq_s[rows, :] = jnp.broadcast_to(
            jnp.sum(q * q, axis=-1, keepdims=True), (TQ, 8))
        cand_s[rows, :] = jnp.full((TQ, K * LW), inf, jnp.float32)

    @pl.when((mi >= 1) & (qi == 0))
    def _prep():
        mem = pm_ref[...]
        mem2 = mem * mem
        ones_r = jnp.ones((8, D), jnp.float32)
        msq_s[...] = jax.lax.dot_general(ones_r, mem2,
                                         (((1,), (1,)), ((), ())),
                                         preferred_element_type=jnp.float32)
        memb_s[...] = mem.astype(jnp.bfloat16)

    @pl.when(mi >= 1)
    def _dist():
        qn2 = qn2_s[rows, :]
        NP = 4 if TM // LW >= 4 else 1
        TS = TM // NP
        chunks = []
        for p in range(NP):
            gp = jax.lax.dot_general(
                qn2, memb_s[pl.ds(p * TS, TS), :], (((1,), (1,)), ((), ())),
                preferred_element_type=jnp.float32)
            chunks += [msq_s[0:1, p * TS + j * LW:p * TS + (j + 1) * LW]
                       + gp[:, j * LW:(j + 1) * LW]
                       for j in range(TS // LW)]
        c5 = _bottomk(chunks, K)
        s = [cand_s[rows, t * LW:(t + 1) * LW] for t in range(K)]
        s = _oem(s, c5)[:K]
        for t in range(K):
            cand_s[rows, t * LW:(t + 1) * LW] = s[t]

        @pl.when(mi == nm1 - 1)
        def _finish():
            cand = jnp.concatenate(s, axis=1)
            mprev = jnp.full((TQ, 1), -inf, jnp.float32)
            cum = jnp.zeros((TQ, 1), jnp.float32)
            tot = jnp.zeros((TQ, 1), jnp.float32)
            kf = jnp.float32(K)
            for _ in range(K):
                cb = jnp.where(cand > mprev, cand, inf)
                m = jnp.min(cb, axis=1, keepdims=True)
                cnt = jnp.sum(jnp.where(cand == m, 1.0, 0.0), axis=1,
                              keepdims=True)
                take = jnp.clip(kf - cum, 0.0, cnt)
                tot = tot + jnp.where(take > 0.0, m * take, 0.0)
                cum = cum + cnt
                mprev = m
            avg = tot / kf + qsq_s[rows, 0:1]
            res = jax.nn.sigmoid(avg - 1.0)
            o_ref[...] = jnp.broadcast_to(res, (TQ, 8))


def kernel(hidden_states, W1, b1, W2, b2, ln_gamma, ln_beta, pattern_memory):
    B, S, d = hidden_states.shape
    BS = B * S
    M = pattern_memory.shape[0]
    x = hidden_states.reshape(BS, d)
    nm = M // TM
    nq = BS // TQ

    out8 = pl.pallas_call(
        _body,
        grid=(nm + 1, nq),
        in_specs=[
            pl.BlockSpec((TQ, d), lambda mi, qi: (jnp.where(mi == 0, qi, 0), 0)),
            pl.BlockSpec((d, 2 * d), lambda mi, qi: (0, 0)),
            pl.BlockSpec((1, 2 * d), lambda mi, qi: (0, 0)),
            pl.BlockSpec((2 * d, d), lambda mi, qi: (0, 0)),
            pl.BlockSpec((1, d), lambda mi, qi: (0, 0)),
            pl.BlockSpec((1, d), lambda mi, qi: (0, 0)),
            pl.BlockSpec((1, d), lambda mi, qi: (0, 0)),
            pl.BlockSpec((TM, d), lambda mi, qi: (jnp.maximum(mi - 1, 0), 0)),
        ],
        out_specs=pl.BlockSpec((TQ, 8), lambda mi, qi: (qi, 0)),
        out_shape=jax.ShapeDtypeStruct((BS, 8), jnp.float32),
        scratch_shapes=[
            pltpu.VMEM((BS, D), jnp.bfloat16),
            pltpu.VMEM((BS, 8), jnp.float32),
            pltpu.VMEM((TM, D), jnp.bfloat16),
            pltpu.VMEM((8, TM), jnp.float32),
            pltpu.VMEM((BS, K * LW), jnp.float32),
        ],
    )(x, W1.astype(jnp.bfloat16), b1.reshape(1, -1),
      W2.astype(jnp.bfloat16), b2.reshape(1, -1),
      ln_gamma.reshape(1, -1), ln_beta.reshape(1, -1), pattern_memory)

    return out8[:, 0].reshape(B, S)
